# initial kernel scaffold (unmeasured)
import jax
import jax.numpy as jnp
from jax import lax
from jax.experimental import pallas as pl
from jax.experimental.pallas import tpu as pltpu


def kernel(
    x,
):
    def body(*refs):
        pass

    out_shape = jax.ShapeDtypeStruct(..., jnp.float32)
    return pl.pallas_call(body, out_shape=out_shape)(...)


# baseline (device time: 11988 ns/iter reference)
import jax
import jax.numpy as jnp
from jax import lax
from jax.experimental import pallas as pl
from jax.experimental.pallas import tpu as pltpu

N_Z = 4


def kernel(x):
    m, n_full = x.shape
    blk = n_full // N_Z

    def body(x_ref, out_ref, xbf_ref, recv_ref, send_sems, recv_sems):
        my_x = lax.axis_index("x")
        my_y = lax.axis_index("y")
        my_z = lax.axis_index("z")

        barrier_sem = pltpu.get_barrier_semaphore()
        for d in range(1, N_Z):
            tgt = lax.rem(my_z + d, N_Z)
            pl.semaphore_signal(
                barrier_sem, inc=1,
                device_id=(my_x, my_y, tgt),
                device_id_type=pl.DeviceIdType.MESH,
            )
        pl.semaphore_wait(barrier_sem, N_Z - 1)

        rdmas = []
        for d in range(1, N_Z):
            tgt = lax.rem(my_z + d, N_Z)
            xbf_ref[d - 1] = x_ref[:, pl.ds(tgt * blk, blk)].astype(
                jnp.bfloat16
            )
            rdma = pltpu.make_async_remote_copy(
                src_ref=xbf_ref.at[d - 1],
                dst_ref=recv_ref.at[d - 1],
                send_sem=send_sems.at[d - 1],
                recv_sem=recv_sems.at[d - 1],
                device_id=(my_x, my_y, tgt),
                device_id_type=pl.DeviceIdType.MESH,
            )
            rdma.start()
            rdmas.append(rdma)

        out_ref[pl.ds(my_z * m, m), :] = x_ref[:, pl.ds(my_z * blk, blk)]

        for d in range(1, N_Z):
            src = lax.rem(my_z + N_Z - d, N_Z)
            rdmas[d - 1].wait_recv()
            out_ref[pl.ds(src * m, m), :] = recv_ref[d - 1].astype(
                jnp.float32
            )

        for d in range(1, N_Z):
            rdmas[d - 1].wait_send()

    out_shape = jax.ShapeDtypeStruct((N_Z * m, blk), jnp.float32)
    return pl.pallas_call(
        body,
        out_shape=out_shape,
        in_specs=[pl.BlockSpec(memory_space=pltpu.VMEM)],
        out_specs=pl.BlockSpec(memory_space=pltpu.VMEM),
        scratch_shapes=[
            pltpu.VMEM((N_Z - 1, m, blk), jnp.bfloat16),
            pltpu.VMEM((N_Z - 1, m, blk), jnp.bfloat16),
            pltpu.SemaphoreType.DMA((N_Z - 1,)),
            pltpu.SemaphoreType.DMA((N_Z - 1,)),
        ],
        compiler_params=pltpu.CompilerParams(collective_id=0),
    )(x)


# device time: 11984 ns/iter; 1.0003x vs baseline; 1.0003x over previous
import jax
import jax.numpy as jnp
from jax import lax
from jax.experimental import pallas as pl
from jax.experimental.pallas import tpu as pltpu

N_Z = 4


def kernel(x):
    m, n_full = x.shape
    blk = n_full // N_Z

    def body(x_ref, out_ref, xbf_ref, recv_ref, send_sems, recv_sems):
        my_x = lax.axis_index("x")
        my_y = lax.axis_index("y")
        my_z = lax.axis_index("z")

        barrier_sem = pltpu.get_barrier_semaphore()
        for d in range(1, N_Z):
            tgt = lax.rem(my_z + d, N_Z)
            pl.semaphore_signal(
                barrier_sem, inc=1,
                device_id=(my_x, my_y, tgt),
                device_id_type=pl.DeviceIdType.MESH,
            )
        for d in range(1, N_Z):
            tgt = lax.rem(my_z + d, N_Z)
            xbf_ref[d - 1] = x_ref[:, pl.ds(tgt * blk, blk)].astype(
                jnp.bfloat16
            )
        pl.semaphore_wait(barrier_sem, N_Z - 1)

        rdmas = []
        for d in range(1, N_Z):
            tgt = lax.rem(my_z + d, N_Z)
            rdma = pltpu.make_async_remote_copy(
                src_ref=xbf_ref.at[d - 1],
                dst_ref=recv_ref.at[d - 1],
                send_sem=send_sems.at[d - 1],
                recv_sem=recv_sems.at[d - 1],
                device_id=(my_x, my_y, tgt),
                device_id_type=pl.DeviceIdType.MESH,
            )
            rdma.start()
            rdmas.append(rdma)

        out_ref[pl.ds(my_z * m, m), :] = x_ref[:, pl.ds(my_z * blk, blk)]

        for d in range(1, N_Z):
            src = lax.rem(my_z + N_Z - d, N_Z)
            rdmas[d - 1].wait_recv()
            out_ref[pl.ds(src * m, m), :] = recv_ref[d - 1].astype(
                jnp.float32
            )

        for d in range(1, N_Z):
            rdmas[d - 1].wait_send()

    out_shape = jax.ShapeDtypeStruct((N_Z * m, blk), jnp.float32)
    return pl.pallas_call(
        body,
        out_shape=out_shape,
        in_specs=[pl.BlockSpec(memory_space=pltpu.VMEM)],
        out_specs=pl.BlockSpec(memory_space=pltpu.VMEM),
        scratch_shapes=[
            pltpu.VMEM((N_Z - 1, m, blk), jnp.bfloat16),
            pltpu.VMEM((N_Z - 1, m, blk), jnp.bfloat16),
            pltpu.SemaphoreType.DMA((N_Z - 1,)),
            pltpu.SemaphoreType.DMA((N_Z - 1,)),
        ],
        compiler_params=pltpu.CompilerParams(collective_id=0),
    )(x)
